# Initial kernel scaffold; baseline (speedup 1.0000x reference)
import jax
import jax.numpy as jnp
from jax.experimental import pallas as pl


def kernel(x, edge_index, W1, b1, W2, b2, W3, b3, W4, b4):
    raise NotImplementedError("write your pallas kernel here")


# speedup vs baseline: 7.9576x; 7.9576x over previous
import functools

import jax
import jax.numpy as jnp
from jax import lax
from jax.experimental import pallas as pl
from jax.experimental.pallas import tpu as pltpu
from jax.experimental.pallas import tpu_sc as plsc

NC = 2
NS = 16
NW = NC * NS
L = 16
D = 128


def _mesh():
    return plsc.VectorSubcoreMesh(
        core_axis_name="c", subcore_axis_name="s",
        num_cores=NC, num_subcores=NS)


def _make_deg(n, e, npad):
    epw = e // NW
    ch = 80
    nchunk = epw // ch
    rpt = npad // NS
    W = 16

    def body(dst_hbm, out_hbm, acc, stage, onesb, didx):
        c = lax.axis_index("c")
        s = lax.axis_index("s")
        wid = s * NC + c
        zeros = jnp.zeros((L,), jnp.float32)
        ones = jnp.ones((L,), jnp.float32)

        @pl.loop(0, rpt)
        def _(r):
            stage[r, :] = zeros

        @pl.loop(0, ch)
        def _(r):
            onesb[r, :] = ones

        pltpu.sync_copy(stage, acc.at[pl.ds(s * rpt, rpt)])
        plsc.subcore_barrier()

        @pl.loop(0, nchunk)
        def _(i):
            base = wid * epw + i * ch
            pltpu.sync_copy(dst_hbm.at[pl.ds(base, ch)], didx)
            pltpu.sync_copy(onesb, acc.at[didx], add=True)

        plsc.subcore_barrier()
        pltpu.sync_copy(acc.at[pl.ds(s * rpt, rpt)], stage)
        pltpu.sync_copy(stage, out_hbm.at[c, pl.ds(s * rpt, rpt)])

    return pl.kernel(
        body,
        out_type=jax.ShapeDtypeStruct((NC, npad, W), jnp.float32),
        mesh=_mesh(),
        scratch_types=[
            pltpu.VMEM_SHARED((npad, W), jnp.float32),
            pltpu.VMEM((rpt, W), jnp.float32),
            pltpu.VMEM((ch, W), jnp.float32),
            pltpu.VMEM((ch,), jnp.int32),
        ])


def _make_scatter(n, e, npad):
    epw = e // NW
    ch = 80
    nchunk = epw // ch
    rpt = npad // NS
    zr = 128

    def body(xs_hbm, src_hbm, dst_hbm, zeros_hbm, out_hbm,
             acc, rows, stage, sidx, didx, gsem):
        c = lax.axis_index("c")
        s = lax.axis_index("s")
        wid = s * NC + c

        pltpu.sync_copy(zeros_hbm, stage)

        @pl.loop(0, rpt // zr)
        def _(k):
            pltpu.sync_copy(stage, acc.at[pl.ds(s * rpt + k * zr, zr)])

        plsc.subcore_barrier()

        @pl.loop(0, nchunk)
        def _(i):
            base = wid * epw + i * ch
            pltpu.sync_copy(src_hbm.at[pl.ds(base, ch)], sidx)
            pltpu.sync_copy(dst_hbm.at[pl.ds(base, ch)], didx)
            pltpu.async_copy(xs_hbm.at[sidx], rows, gsem).wait()
            pltpu.sync_copy(rows, acc.at[didx], add=True)

        plsc.subcore_barrier()

        @pl.loop(0, rpt // zr)
        def _(k):
            r0 = s * rpt + k * zr
            pltpu.sync_copy(acc.at[pl.ds(r0, zr)], stage)
            pltpu.sync_copy(stage, out_hbm.at[c, pl.ds(r0, zr)])

    return pl.kernel(
        body,
        out_type=jax.ShapeDtypeStruct((NC, npad, D), jnp.float32),
        mesh=_mesh(),
        scratch_types=[
            pltpu.VMEM_SHARED((npad, D), jnp.float32),
            pltpu.VMEM((ch, D), jnp.float32),
            pltpu.VMEM((zr, D), jnp.float32),
            pltpu.VMEM((ch,), jnp.int32),
            pltpu.VMEM((ch,), jnp.int32),
            pltpu.SemaphoreType.DMA,
        ])


def _make_edge(n, e):
    epw = e // NW
    ch = 80
    nchunk = epw // ch
    ng = ch // L

    def body(a_hbm, b_hbm, src_hbm, dst_hbm, w4_hbm, out_hbm,
             arows, brows, sidx, didx, w4buf, obuf, gsem):
        c = lax.axis_index("c")
        s = lax.axis_index("s")
        wid = s * NC + c

        pltpu.sync_copy(w4_hbm, w4buf)
        w4 = [w4buf[pl.ds(j * L, L)] for j in range(D // L)]

        @pl.loop(0, nchunk)
        def _(i):
            base = wid * epw + i * ch
            pltpu.sync_copy(src_hbm.at[pl.ds(base, ch)], sidx)
            pltpu.sync_copy(dst_hbm.at[pl.ds(base, ch)], didx)
            ca = pltpu.async_copy(a_hbm.at[sidx], arows, gsem)
            cb = pltpu.async_copy(b_hbm.at[didx], brows, gsem)
            ca.wait()
            cb.wait()

            @pl.loop(0, ng)
            def _(g):
                for ee in range(L):
                    erow = g * L + ee
                    acc = jnp.zeros((L,), jnp.float32)
                    for j in range(D // L):
                        av = arows[erow, pl.ds(j * L, L)]
                        bv = brows[erow, pl.ds(j * L, L)]
                        acc = acc + jnp.maximum(av + bv, 0.0) * w4[j]
                    obuf[g * L + ee, :] = acc

            pltpu.sync_copy(obuf, out_hbm.at[pl.ds(base, ch)])

    return pl.kernel(
        body,
        out_type=jax.ShapeDtypeStruct((e, L), jnp.float32),
        mesh=_mesh(),
        scratch_types=[
            pltpu.VMEM((ch, D), jnp.float32),
            pltpu.VMEM((ch, D), jnp.float32),
            pltpu.VMEM((ch,), jnp.int32),
            pltpu.VMEM((ch,), jnp.int32),
            pltpu.VMEM((D,), jnp.float32),
            pltpu.VMEM((ch, L), jnp.float32),
            pltpu.SemaphoreType.DMA,
        ])


def _tc0_body(deg_ref, x_ref, w1_ref, dis_ref, xs1_ref):
    deg = jnp.sum(deg_ref[...], axis=0) + 1.0
    dis = lax.rsqrt(deg)
    dis_ref[...] = dis
    xs1_ref[...] = dis * jnp.dot(
        x_ref[...], w1_ref[...], preferred_element_type=jnp.float32)


def _tc1_body(acc_ref, xs1_ref, dis_ref, b1_ref, w2_ref, xs2_ref):
    dis = dis_ref[...]
    tot = acc_ref[0] + acc_ref[1] + xs1_ref[...]
    h1 = jnp.maximum(dis * tot + b1_ref[...], 0.0)
    xs2_ref[...] = dis * jnp.dot(
        h1, w2_ref[...], preferred_element_type=jnp.float32)


def _tc2_body(acc_ref, xs2_ref, dis_ref, b2_ref, w3a_ref, w3b_ref, b3_ref,
              h2_ref, a_ref, b_ref):
    dis = dis_ref[...]
    tot = acc_ref[0] + acc_ref[1] + xs2_ref[...]
    h2 = jnp.maximum(dis * tot + b2_ref[...], 0.0)
    h2_ref[...] = h2
    a_ref[...] = jnp.dot(
        h2, w3a_ref[...], preferred_element_type=jnp.float32) + b3_ref[...]
    b_ref[...] = jnp.dot(
        h2, w3b_ref[...], preferred_element_type=jnp.float32)


def _row_spec(bn, d):
    return pl.BlockSpec((bn, d), lambda i: (i, 0))


def _full_spec(shape):
    nz = (0,) * len(shape)
    return pl.BlockSpec(shape, lambda i, _nz=nz: _nz)


def _make_tc(n):
    bn = 2000
    grid = (n // bn,)
    f32 = jnp.float32

    tc0 = pl.pallas_call(
        _tc0_body,
        grid=grid,
        in_specs=[
            pl.BlockSpec((NC, bn, 1), lambda i: (0, i, 0)),
            _row_spec(bn, D),
            _full_spec((D, D)),
        ],
        out_specs=[_row_spec(bn, 1), _row_spec(bn, D)],
        out_shape=[jax.ShapeDtypeStruct((n, 1), f32),
                   jax.ShapeDtypeStruct((n, D), f32)],
    )

    tc1 = pl.pallas_call(
        _tc1_body,
        grid=grid,
        in_specs=[
            pl.BlockSpec((NC, bn, D), lambda i: (0, i, 0)),
            _row_spec(bn, D),
            _row_spec(bn, 1),
            _full_spec((1, D)),
            _full_spec((D, D)),
        ],
        out_specs=[_row_spec(bn, D)],
        out_shape=[jax.ShapeDtypeStruct((n, D), f32)],
    )

    tc2 = pl.pallas_call(
        _tc2_body,
        grid=grid,
        in_specs=[
            pl.BlockSpec((NC, bn, D), lambda i: (0, i, 0)),
            _row_spec(bn, D),
            _row_spec(bn, 1),
            _full_spec((1, D)),
            _full_spec((D, D)),
            _full_spec((D, D)),
            _full_spec((1, D)),
        ],
        out_specs=[_row_spec(bn, D)] * 3,
        out_shape=[jax.ShapeDtypeStruct((n, D), f32)] * 3,
    )
    return tc0, tc1, tc2


def _tc3_body(p_ref, b4_ref, o_ref):
    t = jnp.sum(p_ref[...], axis=-1, keepdims=True) + b4_ref[...]
    o_ref[...] = jax.nn.sigmoid(t)


def _make_tc3(e):
    be = 8000
    return pl.pallas_call(
        _tc3_body,
        grid=(e // be,),
        in_specs=[
            pl.BlockSpec((be, L), lambda i: (i, 0)),
            pl.BlockSpec((1, 1), lambda i: (0, 0)),
        ],
        out_specs=[pl.BlockSpec((be, 1), lambda i: (i, 0))],
        out_shape=[jax.ShapeDtypeStruct((e, 1), jnp.float32)],
    )


@jax.jit
def kernel(x, edge_index, W1, b1, W2, b2, W3, b3, W4, b4):
    n = x.shape[0]
    e = edge_index.shape[1]
    src = edge_index[0]
    dst = edge_index[1]

    npad = ((n + 128 * NS - 1) // (128 * NS)) * (128 * NS)
    deg_k = _make_deg(n, e, npad)
    scat_k = _make_scatter(n, e, npad)
    edge_k = _make_edge(n, e)
    tc0, tc1, tc2 = _make_tc(n)

    zeros_stage = jnp.zeros((128, D), jnp.float32)

    deg2 = deg_k(dst)
    dis, xs1 = tc0(deg2[:, :, :1], x, W1)

    acc1 = scat_k(xs1, src, dst, zeros_stage)[:, :n, :]
    (xs2,) = tc1(acc1, xs1, dis, b1.reshape(1, D), W2)

    acc2 = scat_k(xs2, src, dst, zeros_stage)[:, :n, :]
    h2, ah, bh = tc2(acc2, xs2, dis, b2.reshape(1, D),
                     W3[:D], W3[D:], b3.reshape(1, D))

    w4r = W4.reshape(D)
    partials = edge_k(ah, bh, src, dst, w4r)
    (prob,) = _make_tc3(e)(partials, b4.reshape(1, 1))
    return (prob.reshape(e), h2)
